# Initial kernel scaffold; baseline (speedup 1.0000x reference)
import jax
import jax.numpy as jnp
from jax.experimental import pallas as pl


def kernel(input_ids, embed_in):
    raise NotImplementedError("write your pallas kernel here")


# speedup vs baseline: 1.5264x; 1.5264x over previous
import functools

import jax
import jax.numpy as jnp
from jax import lax
from jax.experimental import pallas as pl
from jax.experimental.pallas import tpu as pltpu
from jax.experimental.pallas import tpu_sc as plsc

NUM_CORES = 2
NUM_SUBCORES = 16
NUM_WORKERS = NUM_CORES * NUM_SUBCORES


@functools.lru_cache(maxsize=None)
def _build_emb_lookup(n_ids: int, vocab: int, embed: int, chunk: int):
    assert n_ids % NUM_WORKERS == 0
    per_w = n_ids // NUM_WORKERS
    assert per_w % chunk == 0
    n_chunks = per_w // chunk
    mesh = plsc.VectorSubcoreMesh(core_axis_name="c", subcore_axis_name="s")

    @functools.partial(
        pl.kernel,
        mesh=mesh,
        out_type=jax.ShapeDtypeStruct((n_ids, embed), jnp.float32),
        scratch_types=[
            pltpu.VMEM((per_w,), jnp.int32),
            pltpu.VMEM((chunk, embed), jnp.float32),
            pltpu.VMEM((chunk, embed), jnp.float32),
            pltpu.SemaphoreType.DMA,
            pltpu.SemaphoreType.DMA,
            pltpu.SemaphoreType.DMA,
            pltpu.SemaphoreType.DMA,
        ],
    )
    def emb(table_hbm, idx_hbm, out_hbm, idx_v, buf0, buf1, gs0, gs1, os0, os1):
        wid = lax.axis_index("s") * NUM_CORES + lax.axis_index("c")
        base = wid * per_w
        pltpu.sync_copy(idx_hbm.at[pl.ds(base, per_w)], idx_v)

        bufs = (buf0, buf1)
        gsems = (gs0, gs1)
        osems = (os0, os1)

        def start_gather(c):
            b = c % 2
            return pltpu.async_copy(
                table_hbm.at[idx_v.at[pl.ds(c * chunk, chunk)]], bufs[b], gsems[b])

        def start_put(c):
            b = c % 2
            return pltpu.async_copy(
                bufs[b], out_hbm.at[pl.ds(base + c * chunk, chunk)], osems[b])

        g = {}
        p = {}
        g[0] = start_gather(0)
        for c in range(n_chunks):
            if c + 1 < n_chunks:
                if c - 1 >= 0:
                    p[c - 1].wait()
                g[c + 1] = start_gather(c + 1)
            g[c].wait()
            p[c] = start_put(c)
        if n_chunks >= 2:
            p[n_chunks - 2].wait()
        p[n_chunks - 1].wait()

    return emb


def kernel(input_ids, embed_in):
    vocab, embed = embed_in.shape
    flat = input_ids.reshape(-1).astype(jnp.int32)
    emb = _build_emb_lookup(flat.shape[0], vocab, embed, chunk=32)
    out = emb(embed_in, flat)
    return out.reshape(input_ids.shape + (embed,))


# speedup vs baseline: 1.5624x; 1.0235x over previous
import functools

import jax
import jax.numpy as jnp
from jax import lax
from jax.experimental import pallas as pl
from jax.experimental.pallas import tpu as pltpu
from jax.experimental.pallas import tpu_sc as plsc

NUM_CORES = 2
NUM_SUBCORES = 16
NUM_WORKERS = NUM_CORES * NUM_SUBCORES


@functools.lru_cache(maxsize=None)
def _build_emb_lookup(n_ids: int, vocab: int, embed: int, chunk: int, nbuf: int):
    assert n_ids % NUM_WORKERS == 0
    per_w = n_ids // NUM_WORKERS
    assert per_w % chunk == 0
    n_chunks = per_w // chunk
    nbuf = min(nbuf, n_chunks)
    mesh = plsc.VectorSubcoreMesh(core_axis_name="c", subcore_axis_name="s")

    @functools.partial(
        pl.kernel,
        mesh=mesh,
        out_type=jax.ShapeDtypeStruct((n_ids, embed), jnp.float32),
        scratch_types=(
            [pltpu.VMEM((per_w,), jnp.int32)]
            + [pltpu.VMEM((chunk, embed), jnp.float32) for _ in range(nbuf)]
            + [pltpu.SemaphoreType.DMA for _ in range(2 * nbuf)]
        ),
    )
    def emb(table_hbm, idx_hbm, out_hbm, idx_v, *scratch):
        bufs = scratch[:nbuf]
        gsems = scratch[nbuf:2 * nbuf]
        osems = scratch[2 * nbuf:]
        wid = lax.axis_index("s") * NUM_CORES + lax.axis_index("c")
        base = wid * per_w
        pltpu.sync_copy(idx_hbm.at[pl.ds(base, per_w)], idx_v)

        def start_gather(c):
            b = c % nbuf
            return pltpu.async_copy(
                table_hbm.at[idx_v.at[pl.ds(c * chunk, chunk)]], bufs[b], gsems[b])

        def start_put(c):
            b = c % nbuf
            return pltpu.async_copy(
                bufs[b], out_hbm.at[pl.ds(base + c * chunk, chunk)], osems[b])

        g = {}
        p = {}
        for c in range(nbuf - 1):
            g[c] = start_gather(c)
        for c in range(n_chunks):
            h = c + nbuf - 1
            if h < n_chunks:
                if c - 1 >= 0:
                    p[c - 1].wait()
                g[h] = start_gather(h)
            g[c].wait()
            p[c] = start_put(c)
        for i in range(max(0, n_chunks - nbuf), n_chunks):
            p[i].wait()

    return emb


def kernel(input_ids, embed_in):
    vocab, embed = embed_in.shape
    flat = input_ids.reshape(-1).astype(jnp.int32)
    emb = _build_emb_lookup(flat.shape[0], vocab, embed, chunk=32, nbuf=3)
    out = emb(embed_in, flat)
    return out.reshape(input_ids.shape + (embed,))


# speedup vs baseline: 1.5852x; 1.0146x over previous
import functools

import jax
import jax.numpy as jnp
from jax import lax
from jax.experimental import pallas as pl
from jax.experimental.pallas import tpu as pltpu
from jax.experimental.pallas import tpu_sc as plsc

NUM_CORES = 2
NUM_SUBCORES = 16
NUM_WORKERS = NUM_CORES * NUM_SUBCORES


@functools.lru_cache(maxsize=None)
def _build_emb_lookup(n_ids: int, vocab: int, embed: int, chunk: int, nbuf: int):
    assert n_ids % NUM_WORKERS == 0
    per_w = n_ids // NUM_WORKERS
    assert per_w % chunk == 0
    n_chunks = per_w // chunk
    nbuf = min(nbuf, n_chunks)
    mesh = plsc.VectorSubcoreMesh(core_axis_name="c", subcore_axis_name="s")

    @functools.partial(
        pl.kernel,
        mesh=mesh,
        out_type=jax.ShapeDtypeStruct((n_ids, embed), jnp.float32),
        scratch_types=(
            [pltpu.VMEM((per_w,), jnp.int32)]
            + [pltpu.VMEM((chunk, embed), jnp.float32) for _ in range(nbuf)]
            + [pltpu.SemaphoreType.DMA for _ in range(2 * nbuf)]
        ),
    )
    def emb(table_hbm, idx_hbm, out_hbm, idx_v, *scratch):
        bufs = scratch[:nbuf]
        gsems = scratch[nbuf:2 * nbuf]
        osems = scratch[2 * nbuf:]
        wid = lax.axis_index("s") * NUM_CORES + lax.axis_index("c")
        base = wid * per_w
        pltpu.sync_copy(idx_hbm.at[pl.ds(base, per_w)], idx_v)

        def start_gather(c):
            b = c % nbuf
            return pltpu.async_copy(
                table_hbm.at[idx_v.at[pl.ds(c * chunk, chunk)]], bufs[b], gsems[b])

        def start_put(c):
            b = c % nbuf
            return pltpu.async_copy(
                bufs[b], out_hbm.at[pl.ds(base + c * chunk, chunk)], osems[b])

        g = {}
        p = {}
        for c in range(nbuf - 1):
            g[c] = start_gather(c)
        for c in range(n_chunks):
            h = c + nbuf - 1
            if h < n_chunks:
                if c - 1 >= 0:
                    p[c - 1].wait()
                g[h] = start_gather(h)
            g[c].wait()
            p[c] = start_put(c)
        for i in range(max(0, n_chunks - nbuf), n_chunks):
            p[i].wait()

    return emb


def kernel(input_ids, embed_in):
    vocab, embed = embed_in.shape
    flat = input_ids.reshape(-1).astype(jnp.int32)
    emb = _build_emb_lookup(flat.shape[0], vocab, embed, chunk=16, nbuf=6)
    out = emb(embed_in, flat)
    return out.reshape(input_ids.shape + (embed,))
